# Initial kernel scaffold; baseline (speedup 1.0000x reference)
import jax
import jax.numpy as jnp
from jax.experimental import pallas as pl


def kernel(h, edge_index, r, norm, emb, W0, loop0, b0, W1, loop1, b1):
    raise NotImplementedError("write your pallas kernel here")


# speedup vs baseline: 5.9115x; 5.9115x over previous
import functools

import jax
import jax.numpy as jnp
from jax import lax
from jax.experimental import pallas as pl
from jax.experimental.pallas import tpu as pltpu
from jax.experimental.pallas import tpu_sc as plsc

NC = 2
NS = 16
NW = NC * NS
LANES = 16
CHUNK = 64


def _block_diag_cat(W):
    R, NB, SUB, _ = W.shape
    D = NB * SUB
    Wbd = jnp.zeros((R, NB, SUB, NB, SUB), W.dtype)
    bidx = jnp.arange(NB)
    Wbd = Wbd.at[:, bidx, :, bidx, :].set(W.transpose(1, 0, 2, 3))
    Wbd = Wbd.reshape(R, D, D)
    return Wbd.transpose(1, 0, 2).reshape(D, R * D)


def _tc_mm0(x, Wf, lw, TN):
    N, D = x.shape
    RD = Wf.shape[1]

    def body(x_ref, wf_ref, lw_ref, y_ref, l_ref):
        xb = x_ref[...]
        y_ref[...] = jnp.dot(xb, wf_ref[...], preferred_element_type=jnp.float32)
        l_ref[...] = jnp.dot(xb, lw_ref[...], preferred_element_type=jnp.float32)

    return pl.pallas_call(
        body,
        grid=(N // TN,),
        in_specs=[
            pl.BlockSpec((TN, D), lambda i: (i, 0)),
            pl.BlockSpec((D, RD), lambda i: (0, 0)),
            pl.BlockSpec((D, D), lambda i: (0, 0)),
        ],
        out_specs=[
            pl.BlockSpec((TN, RD), lambda i: (i, 0)),
            pl.BlockSpec((TN, D), lambda i: (i, 0)),
        ],
        out_shape=[
            jax.ShapeDtypeStruct((N, RD), jnp.float32),
            jax.ShapeDtypeStruct((N, D), jnp.float32),
        ],
    )(x, Wf, lw)


def _tc_mm_fused(agg, l_prev, b_prev, Wf, lw, TN):
    _, N, D = agg.shape
    RD = Wf.shape[1]

    def body(a_ref, lp_ref, b_ref, wf_ref, lw_ref, y_ref, l_ref):
        xb = a_ref[0] + a_ref[1] + lp_ref[...] + b_ref[...]
        xb = jnp.maximum(xb, 0.0)
        y_ref[...] = jnp.dot(xb, wf_ref[...], preferred_element_type=jnp.float32)
        l_ref[...] = jnp.dot(xb, lw_ref[...], preferred_element_type=jnp.float32)

    return pl.pallas_call(
        body,
        grid=(N // TN,),
        in_specs=[
            pl.BlockSpec((2, TN, D), lambda i: (0, i, 0)),
            pl.BlockSpec((TN, D), lambda i: (i, 0)),
            pl.BlockSpec((1, D), lambda i: (0, 0)),
            pl.BlockSpec((D, RD), lambda i: (0, 0)),
            pl.BlockSpec((D, D), lambda i: (0, 0)),
        ],
        out_specs=[
            pl.BlockSpec((TN, RD), lambda i: (i, 0)),
            pl.BlockSpec((TN, D), lambda i: (i, 0)),
        ],
        out_shape=[
            jax.ShapeDtypeStruct((N, RD), jnp.float32),
            jax.ShapeDtypeStruct((N, D), jnp.float32),
        ],
    )(agg, l_prev, b_prev.reshape(1, D), Wf, lw)


def _tc_final(agg, l_prev, b_prev, TN):
    _, N, D = agg.shape

    def body(a_ref, lp_ref, b_ref, o_ref):
        o_ref[...] = a_ref[0] + a_ref[1] + lp_ref[...] + b_ref[...]

    return pl.pallas_call(
        body,
        grid=(N // TN,),
        in_specs=[
            pl.BlockSpec((2, TN, D), lambda i: (0, i, 0)),
            pl.BlockSpec((TN, D), lambda i: (i, 0)),
            pl.BlockSpec((1, D), lambda i: (0, 0)),
        ],
        out_specs=pl.BlockSpec((TN, D), lambda i: (i, 0)),
        out_shape=jax.ShapeDtypeStruct((N, D), jnp.float32),
    )(agg, l_prev, b_prev.reshape(1, D))


def _sc_edge_pass(Y, gidx, dst, norm, N):
    NR, D = Y.shape
    Ep = gidx.shape[0]
    chunks_per_worker = Ep // (NW * CHUNK)
    rows_per_tile = N // NS
    ZR = 64
    assert rows_per_tile % ZR == 0 and N % NS == 0

    mesh = plsc.VectorSubcoreMesh(core_axis_name="c", subcore_axis_name="s")

    @functools.partial(
        pl.kernel,
        out_type=jax.ShapeDtypeStruct((NC, N, D), jnp.float32),
        mesh=mesh,
        scratch_types=[
            pltpu.VMEM_SHARED((N, D), jnp.float32),
            pltpu.VMEM((CHUNK,), jnp.int32),
            pltpu.VMEM((CHUNK,), jnp.int32),
            pltpu.VMEM((CHUNK * LANES,), jnp.float32),
            pltpu.VMEM((CHUNK, D), jnp.float32),
            pltpu.VMEM((ZR, D), jnp.float32),
            pltpu.SemaphoreType.DMA,
        ],
    )
    def k(y_hbm, gidx_hbm, dst_hbm, norm_hbm, out_hbm,
          agg_sh, idx_v, dst_v, norm_v, rows_v, zeros_v, sem):
        cid = lax.axis_index("c")
        sid = lax.axis_index("s")
        wid = sid * NC + cid

        def zrow(i, _):
            for q in range(D // LANES):
                zeros_v[i, pl.ds(q * LANES, LANES)] = jnp.zeros((LANES,), jnp.float32)
            return 0
        lax.fori_loop(0, ZR, zrow, 0)
        for t in range(rows_per_tile // ZR):
            pltpu.sync_copy(zeros_v, agg_sh.at[pl.ds(sid * rows_per_tile + t * ZR, ZR)])
        plsc.subcore_barrier()

        def chunk_body(kk, _):
            base = (kk * NW + wid) * CHUNK
            pltpu.sync_copy(gidx_hbm.at[pl.ds(base, CHUNK)], idx_v)
            pltpu.sync_copy(dst_hbm.at[pl.ds(base, CHUNK)], dst_v)
            pltpu.sync_copy(norm_hbm.at[pl.ds(base * LANES, CHUNK * LANES)], norm_v)
            pltpu.async_copy(y_hbm.at[idx_v], rows_v, sem).wait()
            for e in range(CHUNK):
                nsplat = norm_v[pl.ds(e * LANES, LANES)]
                for q in range(D // LANES):
                    sl = pl.ds(q * LANES, LANES)
                    rows_v[e, sl] = rows_v[e, sl] * nsplat
            pltpu.sync_copy(rows_v, agg_sh.at[dst_v], add=True)
            return 0
        lax.fori_loop(0, chunks_per_worker, chunk_body, 0)
        plsc.subcore_barrier()

        pltpu.sync_copy(
            agg_sh.at[pl.ds(sid * rows_per_tile, rows_per_tile)],
            out_hbm.at[cid, pl.ds(sid * rows_per_tile, rows_per_tile), :])

    return k(Y, gidx, dst, norm)


def kernel(h, edge_index, r, norm, emb, W0, loop0, b0, W1, loop1, b1):
    N, D = emb.shape
    E = edge_index.shape[1]
    R = W0.shape[0]
    Np = ((N + NS * 64 - 1) // (NS * 64)) * (NS * 64)
    TN = 512
    assert Np % TN == 0

    x0 = jnp.take(emb, h, axis=0)
    x0 = jnp.pad(x0, ((0, Np - N), (0, 0)))

    src = edge_index[0]
    dst = edge_index[1]
    gidx = src * R + r
    Ep = ((E + NW * CHUNK - 1) // (NW * CHUNK)) * (NW * CHUNK)
    pad = Ep - E
    gidx_p = jnp.concatenate([gidx, jnp.zeros((pad,), jnp.int32)])
    dst_p = jnp.concatenate([dst, jnp.zeros((pad,), jnp.int32)])
    norm_p = jnp.concatenate([norm[:, 0], jnp.zeros((pad,), jnp.float32)])
    norm16 = jnp.broadcast_to(norm_p[:, None], (Ep, LANES)).reshape(Ep * LANES)

    Wf0 = _block_diag_cat(W0)
    Wf1 = _block_diag_cat(W1)

    Y0, L0 = _tc_mm0(x0, Wf0, loop0, TN)
    agg0 = _sc_edge_pass(Y0.reshape(Np * R, D), gidx_p, dst_p, norm16, Np)
    Y1, L1 = _tc_mm_fused(agg0, L0, b0, Wf1, loop1, TN)
    agg1 = _sc_edge_pass(Y1.reshape(Np * R, D), gidx_p, dst_p, norm16, Np)
    return _tc_final(agg1, L1, b1, TN)[:N]


# speedup vs baseline: 5.9145x; 1.0005x over previous
import functools

import jax
import jax.numpy as jnp
from jax import lax
from jax.experimental import pallas as pl
from jax.experimental.pallas import tpu as pltpu
from jax.experimental.pallas import tpu_sc as plsc

NC = 2
NS = 16
NW = NC * NS
LANES = 16
CHUNK = 64
NSLOT = 3


def _block_diag_cat(W):
    R, NB, SUB, _ = W.shape
    D = NB * SUB
    Wbd = jnp.zeros((R, NB, SUB, NB, SUB), W.dtype)
    bidx = jnp.arange(NB)
    Wbd = Wbd.at[:, bidx, :, bidx, :].set(W.transpose(1, 0, 2, 3))
    Wbd = Wbd.reshape(R, D, D)
    return Wbd.transpose(1, 0, 2).reshape(D, R * D)


def _tc_mm0(x, Wf, lw, TN):
    N, D = x.shape
    RD = Wf.shape[1]

    def body(x_ref, wf_ref, lw_ref, y_ref, l_ref):
        xb = x_ref[...]
        y_ref[...] = jnp.dot(xb, wf_ref[...], preferred_element_type=jnp.float32)
        l_ref[...] = jnp.dot(xb, lw_ref[...], preferred_element_type=jnp.float32)

    return pl.pallas_call(
        body,
        grid=(N // TN,),
        in_specs=[
            pl.BlockSpec((TN, D), lambda i: (i, 0)),
            pl.BlockSpec((D, RD), lambda i: (0, 0)),
            pl.BlockSpec((D, D), lambda i: (0, 0)),
        ],
        out_specs=[
            pl.BlockSpec((TN, RD), lambda i: (i, 0)),
            pl.BlockSpec((TN, D), lambda i: (i, 0)),
        ],
        out_shape=[
            jax.ShapeDtypeStruct((N, RD), jnp.float32),
            jax.ShapeDtypeStruct((N, D), jnp.float32),
        ],
    )(x, Wf, lw)


def _tc_mm_fused(agg, l_prev, b_prev, Wf, lw, TN):
    _, N, D = agg.shape
    RD = Wf.shape[1]

    def body(a_ref, lp_ref, b_ref, wf_ref, lw_ref, y_ref, l_ref):
        xb = a_ref[0] + a_ref[1] + lp_ref[...] + b_ref[...]
        xb = jnp.maximum(xb, 0.0)
        y_ref[...] = jnp.dot(xb, wf_ref[...], preferred_element_type=jnp.float32)
        l_ref[...] = jnp.dot(xb, lw_ref[...], preferred_element_type=jnp.float32)

    return pl.pallas_call(
        body,
        grid=(N // TN,),
        in_specs=[
            pl.BlockSpec((2, TN, D), lambda i: (0, i, 0)),
            pl.BlockSpec((TN, D), lambda i: (i, 0)),
            pl.BlockSpec((1, D), lambda i: (0, 0)),
            pl.BlockSpec((D, RD), lambda i: (0, 0)),
            pl.BlockSpec((D, D), lambda i: (0, 0)),
        ],
        out_specs=[
            pl.BlockSpec((TN, RD), lambda i: (i, 0)),
            pl.BlockSpec((TN, D), lambda i: (i, 0)),
        ],
        out_shape=[
            jax.ShapeDtypeStruct((N, RD), jnp.float32),
            jax.ShapeDtypeStruct((N, D), jnp.float32),
        ],
    )(agg, l_prev, b_prev.reshape(1, D), Wf, lw)


def _tc_final(agg, l_prev, b_prev, TN):
    _, N, D = agg.shape

    def body(a_ref, lp_ref, b_ref, o_ref):
        o_ref[...] = a_ref[0] + a_ref[1] + lp_ref[...] + b_ref[...]

    return pl.pallas_call(
        body,
        grid=(N // TN,),
        in_specs=[
            pl.BlockSpec((2, TN, D), lambda i: (0, i, 0)),
            pl.BlockSpec((TN, D), lambda i: (i, 0)),
            pl.BlockSpec((1, D), lambda i: (0, 0)),
        ],
        out_specs=pl.BlockSpec((TN, D), lambda i: (i, 0)),
        out_shape=jax.ShapeDtypeStruct((N, D), jnp.float32),
    )(agg, l_prev, b_prev.reshape(1, D))


def _sc_edge_pass(Y, gidx, dst, norm, N):
    NR, D = Y.shape
    Ep = gidx.shape[0]
    nk = Ep // (NW * CHUNK)
    epw = nk * CHUNK
    CL = CHUNK * LANES
    rows_per_tile = N // NS
    ZR = 32
    assert rows_per_tile % ZR == 0 and N % NS == 0
    assert nk % NSLOT == 0 and NSLOT == 3 and epw % 8 == 0

    mesh = plsc.VectorSubcoreMesh(core_axis_name="c", subcore_axis_name="s")

    @functools.partial(
        pl.kernel,
        out_type=jax.ShapeDtypeStruct((NC, N, D), jnp.float32),
        mesh=mesh,
        scratch_types=[
            pltpu.VMEM_SHARED((N, D), jnp.float32),
            pltpu.VMEM((epw,), jnp.int32),
            pltpu.VMEM((epw,), jnp.int32),
            pltpu.VMEM((CL,), jnp.float32),
            pltpu.VMEM((CL,), jnp.float32),
            pltpu.VMEM((CL,), jnp.float32),
            pltpu.VMEM((NSLOT, CHUNK, D), jnp.float32),
            pltpu.VMEM((ZR, D), jnp.float32),
            pltpu.SemaphoreType.DMA((NSLOT,)),
            pltpu.SemaphoreType.DMA((NSLOT,)),
            pltpu.SemaphoreType.DMA((NSLOT,)),
        ],
    )
    def k(y_hbm, gidx_hbm, dst_hbm, norm_hbm, out_hbm,
          agg_sh, idx_all, dst_all, norm_v0, norm_v1, norm_v2, rows_v,
          zeros_v, sem_g, sem_n, sem_s):
        cid = lax.axis_index("c")
        sid = lax.axis_index("s")
        wid = sid * NC + cid
        norm_slots = (norm_v0, norm_v1, norm_v2)

        def zrow(i, _):
            for q in range(D // LANES):
                zeros_v[i, pl.ds(q * LANES, LANES)] = jnp.zeros((LANES,), jnp.float32)
            return 0
        lax.fori_loop(0, ZR, zrow, 0)
        for t in range(rows_per_tile // ZR):
            pltpu.sync_copy(zeros_v, agg_sh.at[pl.ds(sid * rows_per_tile + t * ZR, ZR)])

        pltpu.sync_copy(gidx_hbm.at[pl.ds(wid * epw, epw)], idx_all)
        pltpu.sync_copy(dst_hbm.at[pl.ds(wid * epw, epw)], dst_all)
        plsc.subcore_barrier()

        def issue(kk, s):
            pltpu.async_copy(y_hbm.at[idx_all.at[pl.ds(kk * CHUNK, CHUNK)]],
                             rows_v.at[s], sem_g.at[s])
            pltpu.async_copy(norm_hbm.at[pl.ds((wid * nk + kk) * CL, CL)],
                             norm_slots[s], sem_n.at[s])

        def wait_in(kk, s):
            pltpu.make_async_copy(y_hbm.at[idx_all.at[pl.ds(kk * CHUNK, CHUNK)]],
                                  rows_v.at[s], sem_g.at[s]).wait()
            pltpu.make_async_copy(norm_hbm.at[pl.ds((wid * nk + kk) * CL, CL)],
                                  norm_slots[s], sem_n.at[s]).wait()

        def issue_scatter(kk, s):
            pltpu.async_copy(rows_v.at[s],
                             agg_sh.at[dst_all.at[pl.ds(kk * CHUNK, CHUNK)]],
                             sem_s.at[s], add=True)

        def wait_scatter(kk, s):
            pltpu.make_async_copy(rows_v.at[s],
                                  agg_sh.at[dst_all.at[pl.ds(kk * CHUNK, CHUNK)]],
                                  sem_s.at[s]).wait()

        UNROLL = 8

        def scale(s):
            def body_e(ei, _):
                for du in range(UNROLL):
                    e = ei * UNROLL + du
                    nsplat = norm_slots[s][pl.ds(e * LANES, LANES)]
                    for q in range(D // LANES):
                        sl = pl.ds(q * LANES, LANES)
                        rows_v[s, e, sl] = rows_v[s, e, sl] * nsplat
                return 0
            lax.fori_loop(0, CHUNK // UNROLL, body_e, 0)

        issue(0, 0)
        issue(1, 1)
        nj = nk // NSLOT

        def body_j(j, _):
            for u in range(NSLOT):
                kk = j * NSLOT + u
                sp = (u + 2) % NSLOT
                wait_in(kk, u)
                scale(u)
                issue_scatter(kk, u)
                if u == 0:
                    @pl.when(j >= 1)
                    def _():
                        wait_scatter(kk - 1, sp)
                    issue(kk + 2, sp)
                else:
                    wait_scatter(kk - 1, sp)

                    @pl.when(j <= nj - 2)
                    def _():
                        issue(kk + 2, sp)
            return 0
        lax.fori_loop(0, nj, body_j, 0)
        wait_scatter(nk - 1, (nk - 1) % NSLOT)
        plsc.subcore_barrier()

        pltpu.sync_copy(
            agg_sh.at[pl.ds(sid * rows_per_tile, rows_per_tile)],
            out_hbm.at[cid, pl.ds(sid * rows_per_tile, rows_per_tile), :])

    return k(Y, gidx, dst, norm)


def kernel(h, edge_index, r, norm, emb, W0, loop0, b0, W1, loop1, b1):
    N, D = emb.shape
    E = edge_index.shape[1]
    R = W0.shape[0]
    Np = ((N + NS * 64 - 1) // (NS * 64)) * (NS * 64)
    TN = 512
    assert Np % TN == 0

    x0 = jnp.take(emb, h, axis=0)
    x0 = jnp.pad(x0, ((0, Np - N), (0, 0)))

    src = edge_index[0]
    dst = edge_index[1]
    gidx = src * R + r
    nk = (E + NW * CHUNK - 1) // (NW * CHUNK)
    nk = ((nk + NSLOT - 1) // NSLOT) * NSLOT
    Ep = nk * NW * CHUNK
    pad = Ep - E
    gidx_p = jnp.concatenate([gidx, jnp.zeros((pad,), jnp.int32)])
    dst_p = jnp.concatenate([dst, jnp.zeros((pad,), jnp.int32)])
    norm_p = jnp.concatenate([norm[:, 0], jnp.zeros((pad,), jnp.float32)])
    norm16 = jnp.broadcast_to(norm_p[:, None], (Ep, LANES)).reshape(Ep * LANES)

    Wf0 = _block_diag_cat(W0)
    Wf1 = _block_diag_cat(W1)

    Y0, L0 = _tc_mm0(x0, Wf0, loop0, TN)
    agg0 = _sc_edge_pass(Y0.reshape(Np * R, D), gidx_p, dst_p, norm16, Np)
    Y1, L1 = _tc_mm_fused(agg0, L0, b0, Wf1, loop1, TN)
    agg1 = _sc_edge_pass(Y1.reshape(Np * R, D), gidx_p, dst_p, norm16, Np)
    return _tc_final(agg1, L1, b1, TN)[:N]
